# baseline (speedup 1.0000x reference)
import functools

import jax
import jax.numpy as jnp
from jax import lax
from jax.experimental import pallas as pl
from jax.experimental.pallas import tpu as pltpu
from jax.experimental.pallas import tpu_sc as plsc

_THRESHOLD = 0.5
_NEG_POS_RATIO = 3
_ALPHA = 1.0
_B, _N, _C = 64, 8732, 81
_NP = _N + 4
_NP4 = _NP * 4
_NVC = _NP // 16


def _loc_body(pd_hbm, td_hbm, cls_hbm, out_hbm, p_v, t_v, c_v, acc_v):
    wid = lax.axis_index("s") * 2 + lax.axis_index("c")
    for rep in range(2):
        b = wid * 2 + rep
        pltpu.sync_copy(pd_hbm.at[b], p_v)
        pltpu.sync_copy(td_hbm.at[b], t_v)
        pltpu.sync_copy(cls_hbm.at[b], c_v)

        acc = jnp.zeros((16,), jnp.float32)
        for comp in range(4):
            def step(j, a, comp=comp):
                off = comp * _NP + j * 16
                pv = p_v[pl.ds(off, 16)]
                tv = t_v[pl.ds(off, 16)]
                cv = c_v[pl.ds(j * 16, 16)]
                ad = jnp.abs(pv - tv)
                s1 = jnp.where(ad < 1.0, 0.5 * ad * ad, ad - 0.5)
                return a + jnp.where(cv != 0, s1, 0.0)

            acc = lax.fori_loop(0, _NVC, step, acc)
        acc_v[...] = acc
        pltpu.sync_copy(acc_v, out_hbm.at[b])


def _loc_call():
    return functools.partial(
        pl.kernel,
        mesh=plsc.VectorSubcoreMesh(core_axis_name="c", subcore_axis_name="s"),
        out_type=jax.ShapeDtypeStruct((_B, 16), jnp.float32),
        scratch_types=[
            pltpu.VMEM((_NP4,), jnp.float32),
            pltpu.VMEM((_NP4,), jnp.float32),
            pltpu.VMEM((_NP,), jnp.int32),
            pltpu.VMEM((16,), jnp.float32),
        ],
    )(_loc_body)


def _conf_body(cls_ref, s_ref, out_ref, acc_ref):
    b = pl.program_id(0)

    @pl.when(b == 0)
    def _init():
        acc_ref[0] = 0.0
        acc_ref[1] = 0.0
        acc_ref[2] = 0.0

    s = s_ref[0].astype(jnp.bfloat16)
    e = jnp.exp(s)
    lse = jnp.log(jnp.sum(e, axis=1).astype(jnp.float32))

    cls = cls_ref[0, 0]
    clsb = cls.astype(jnp.int16)
    col = jax.lax.broadcasted_iota(jnp.int16, (_N, _C), 1)
    s_true = jnp.sum(
        jnp.where(col == clsb[:, None], s, jnp.bfloat16(0.0)), axis=1
    ).astype(jnp.float32)
    ce = lse - s_true

    pos = cls != 0
    posf = pos.astype(jnp.float32)
    npos = jnp.sum(posf)
    conf_pos = jnp.sum(ce * posf)
    ce_neg = jnp.where(pos, 0.0, ce)
    sum_neg = jnp.sum(ce_neg)

    acc_ref[0] = acc_ref[0] + conf_pos
    acc_ref[2] = acc_ref[2] + npos

    k_f = jnp.float32(_NEG_POS_RATIO) * npos
    n_neg = jnp.float32(_N) - npos
    fast = k_f >= n_neg

    @pl.when(fast)
    def _all_negatives():
        acc_ref[1] = acc_ref[1] + sum_neg

    @pl.when(jnp.logical_not(fast))
    def _topk():
        def step(i, rb):
            cand = rb | (jnp.int32(1) << (30 - i))
            t = jax.lax.bitcast_convert_type(cand, jnp.float32)
            cnt = jnp.sum(jnp.where(ce_neg >= t, 1.0, 0.0))
            return jnp.where(cnt >= k_f, cand, rb)

        rb = jax.lax.fori_loop(0, 31, step, jnp.int32(0))
        t = jax.lax.bitcast_convert_type(rb, jnp.float32)
        gt = ce_neg > t
        cnt_gt = jnp.sum(gt.astype(jnp.float32))
        sum_gt = jnp.sum(jnp.where(gt, ce_neg, 0.0))
        acc_ref[1] = acc_ref[1] + jnp.where(
            k_f > 0.0, sum_gt + (k_f - cnt_gt) * t, 0.0
        )

    @pl.when(b == _B - 1)
    def _finish():
        out_ref[0] = acc_ref[0] + acc_ref[1]
        out_ref[1] = acc_ref[2]


def _conf_call(interpret=False):
    return pl.pallas_call(
        _conf_body,
        grid=(_B,),
        in_specs=[
            pl.BlockSpec((1, 1, _N), lambda b: (b, 0, 0)),
            pl.BlockSpec((1, _N, _C), lambda b: (b, 0, 0)),
        ],
        out_specs=pl.BlockSpec(memory_space=pltpu.SMEM),
        out_shape=jax.ShapeDtypeStruct((2,), jnp.float32),
        scratch_shapes=[pltpu.SMEM((3,), jnp.float32)],
        interpret=interpret,
    )


def _combine_body(conf_ref, loc_ref, out_ref):
    npos_t = conf_ref[1]
    denom = jnp.maximum(npos_t, 1.0)
    loc = jnp.sum(loc_ref[...])
    out_ref[0] = conf_ref[0] / denom + _ALPHA * loc / denom


def _combine_call(interpret=False):
    return pl.pallas_call(
        _combine_body,
        in_specs=[
            pl.BlockSpec(memory_space=pltpu.SMEM),
            pl.BlockSpec(),
        ],
        out_specs=pl.BlockSpec(memory_space=pltpu.SMEM),
        out_shape=jax.ShapeDtypeStruct((1,), jnp.float32),
        interpret=interpret,
    )


def kernel(predicted_locs, predicted_scores, true_locs, true_classes):
    cls3 = true_classes.reshape(_B, 1, _N)
    pd_f = jnp.pad(
        jnp.swapaxes(predicted_locs, 1, 2), ((0, 0), (0, 0), (0, _NP - _N))
    ).reshape(_B, _NP4)
    td_f = jnp.pad(
        jnp.swapaxes(true_locs, 1, 2), ((0, 0), (0, 0), (0, _NP - _N))
    ).reshape(_B, _NP4)
    cls_pad = jnp.pad(true_classes, ((0, 0), (0, _NP - _N)))
    loc_part = _loc_call()(pd_f, td_f, cls_pad)
    conf_part = _conf_call()(cls3, predicted_scores)
    out = _combine_call()(conf_part, loc_part)
    return out[0]
